# Initial kernel scaffold; baseline (speedup 1.0000x reference)
import jax
import jax.numpy as jnp
from jax.experimental import pallas as pl


def kernel(seq_length, table):
    raise NotImplementedError("write your pallas kernel here")


# speedup vs baseline: 6.3965x; 6.3965x over previous
import functools

import jax
import jax.numpy as jnp
from jax import lax
from jax.experimental import pallas as pl
from jax.experimental.pallas import tpu as pltpu
from jax.experimental.pallas import tpu_sc as plsc

S = 2048
HD = 64
T = 2 * S + 1
ROW = S * HD
NC = 2
NS = 16
NW = NC * NS
RPW = S // NW
HALF = S // 2
WIN = (HALF + RPW) * HD
CHUNK = HALF * HD

_mesh = plsc.VectorSubcoreMesh(core_axis_name="c", subcore_axis_name="s")


@functools.partial(
    pl.kernel,
    mesh=_mesh,
    out_type=jax.ShapeDtypeStruct((S * ROW,), jnp.float32),
    scratch_types=[
        pltpu.VMEM((WIN,), jnp.float32),
        pltpu.SemaphoreType.DMA,
    ],
)
def _relpos_rows(table_hbm, out_hbm, win, sem):
    c = lax.axis_index("c")
    s = lax.axis_index("s")
    wid = s * NC + c
    i0 = wid * RPW

    def half(h, carry):
        w0 = (S - i0 - (RPW - 1) + h * HALF) * HD
        pltpu.sync_copy(table_hbm.at[pl.ds(w0, WIN)], win)

        def row(r, inner):
            i = i0 + r
            src = (RPW - 1 - r) * HD
            dst = i * ROW + h * CHUNK
            pltpu.sync_copy(win.at[pl.ds(src, CHUNK)], out_hbm.at[pl.ds(dst, CHUNK)])
            return inner

        return lax.fori_loop(0, RPW, row, carry)

    lax.fori_loop(0, 2, half, 0)


def kernel(seq_length, table):
    del seq_length
    flat = table.reshape(T * HD)
    out = _relpos_rows(flat)
    return out.reshape(S, S, HD)
